# baseline (speedup 1.0000x reference)
import jax
import jax.numpy as jnp
from jax import lax
from jax.experimental import pallas as pl
from jax.experimental.pallas import tpu as pltpu
from jax.experimental.pallas import tpu_sc as plsc

NUM_FIELDS = 26
FIELD_SIZE = 40000
BATCH = 16384
L = 16
NC = 2
NS = 16
NW = NC * NS
B_PER_W = BATCH // NW
NPIECE = 2
FPP = NUM_FIELDS // NPIECE

_SC_PARAMS = pltpu.CompilerParams(needs_layout_passes=False)


def _mesh():
    return plsc.VectorSubcoreMesh(core_axis_name="c", subcore_axis_name="s")


def _wid():
    return lax.axis_index("s") * NC + lax.axis_index("c")


def _transpose_body(x_hbm, xt_hbm, xb_v, xtl_v):
    w = _wid()
    rowbase = pl.multiple_of(w * B_PER_W, 8)
    pltpu.sync_copy(x_hbm.at[pl.ds(rowbase, B_PER_W), :], xb_v)
    iota = lax.iota(jnp.int32, L)

    def chunk(g, carry):
        rows = iota + g * L
        for f in range(NUM_FIELDS):
            xtl_v[f, pl.ds(g * L, L)] = plsc.load_gather(xb_v, [rows, iota * 0 + f])
        return carry

    lax.fori_loop(0, B_PER_W // L, chunk, 0)
    pltpu.sync_copy(xtl_v, xt_hbm.at[w])


def _gather_body(p0_hbm, p1_hbm, xt_hbm, partials_hbm, tab_v, xrow_v, vals_v):
    w = _wid()

    for f in range(NUM_FIELDS):

        @pl.when(w == f)
        def _(f=f):
            src = (p0_hbm, p1_hbm)[f // FPP]
            pltpu.sync_copy(
                src.at[pl.ds((f % FPP) * FIELD_SIZE, FIELD_SIZE)], tab_v
            )
            for t in range(NW):
                pltpu.sync_copy(
                    xt_hbm.at[t, f], xrow_v.at[pl.ds(t * B_PER_W, B_PER_W)]
                )

            def gath(i, carry):
                sl = pl.ds(i * L, L)
                vals_v[sl] = plsc.load_gather(tab_v, [xrow_v[sl]])
                return carry

            lax.fori_loop(0, BATCH // L, gath, 0)
            pltpu.sync_copy(vals_v, partials_hbm.at[f])


def _reduce_body(partials_hbm, bias_hbm, out_hbm, pcols_v, bias_v, out_v):
    w = _wid()
    base = pl.multiple_of(w * B_PER_W, 8)
    for f in range(NUM_FIELDS):
        pltpu.sync_copy(partials_hbm.at[f, pl.ds(base, B_PER_W)], pcols_v.at[f])
    pltpu.sync_copy(bias_hbm, bias_v)
    iota = lax.iota(jnp.int32, L)
    bias_b = plsc.load_gather(bias_v, [iota * 0])

    def red(q, carry):
        acc = bias_b
        for f in range(NUM_FIELDS):
            acc = acc + pcols_v[f, pl.ds(q * L, L)]
        out_v[pl.ds(q * L, L)] = acc
        return carry

    lax.fori_loop(0, B_PER_W // L, red, 0)
    pltpu.sync_copy(out_v, out_hbm.at[pl.ds(base, B_PER_W)])


@jax.jit
def _run(x, bias, p0, p1):
    k1 = pl.kernel(
        _transpose_body,
        out_type=jax.ShapeDtypeStruct((NW, NUM_FIELDS, B_PER_W), jnp.int32),
        mesh=_mesh(),
        scratch_types=[
            pltpu.VMEM((B_PER_W, NUM_FIELDS), jnp.int32),
            pltpu.VMEM((NUM_FIELDS, B_PER_W), jnp.int32),
        ],
        name="fl_transpose",
        compiler_params=_SC_PARAMS,
    )
    xt = k1(x)

    k2 = pl.kernel(
        _gather_body,
        out_type=jax.ShapeDtypeStruct((NUM_FIELDS, BATCH), jnp.float32),
        mesh=_mesh(),
        scratch_types=[
            pltpu.VMEM((FIELD_SIZE,), jnp.float32),
            pltpu.VMEM((BATCH,), jnp.int32),
            pltpu.VMEM((BATCH,), jnp.float32),
        ],
        name="fl_gather",
        compiler_params=_SC_PARAMS,
    )
    partials = k2(p0, p1, xt)

    k3 = pl.kernel(
        _reduce_body,
        out_type=jax.ShapeDtypeStruct((BATCH,), jnp.float32),
        mesh=_mesh(),
        scratch_types=[
            pltpu.VMEM((NUM_FIELDS, B_PER_W), jnp.float32),
            pltpu.VMEM((1,), jnp.float32),
            pltpu.VMEM((B_PER_W,), jnp.float32),
        ],
        name="fl_reduce",
        compiler_params=_SC_PARAMS,
    )
    return k3(partials, bias)


def kernel(x, fc_weight, bias):
    psz = FPP * FIELD_SIZE
    p0 = fc_weight[:psz, 0]
    p1 = fc_weight[psz:, 0]
    out = _run(x.astype(jnp.int32), bias.astype(jnp.float32), p0, p1)
    return out.reshape(BATCH, 1)


# speedup vs baseline: 1.3825x; 1.3825x over previous
import jax
import jax.numpy as jnp
from jax import lax
from jax.experimental import pallas as pl
from jax.experimental.pallas import tpu as pltpu
from jax.experimental.pallas import tpu_sc as plsc

NUM_FIELDS = 26
FIELD_SIZE = 40000
BATCH = 16384
L = 16
NC = 2
NS = 16
NW = NC * NS
B_PER_W = BATCH // NW
NPIECE = 2
FPP = NUM_FIELDS // NPIECE

_SC_PARAMS = pltpu.CompilerParams(needs_layout_passes=False)


def _mesh():
    return plsc.VectorSubcoreMesh(core_axis_name="c", subcore_axis_name="s")


def _wid():
    return lax.axis_index("s") * NC + lax.axis_index("c")


def _transpose_body(x_hbm, xt_hbm, xb_v, xtl_v):
    w = _wid()
    rowbase = pl.multiple_of(w * B_PER_W, 8)
    pltpu.sync_copy(x_hbm.at[pl.ds(rowbase, B_PER_W), :], xb_v)
    iota = lax.iota(jnp.int32, L)

    def chunk(g, carry):
        rows = iota + g * L
        for f in range(NUM_FIELDS):
            xtl_v[f, pl.ds(g * L, L)] = plsc.load_gather(xb_v, [rows, iota * 0 + f])
        return carry

    lax.fori_loop(0, B_PER_W // L, chunk, 0)
    pltpu.sync_copy(xtl_v, xt_hbm.at[w])


def _gather_body(p0_hbm, p1_hbm, xt_hbm, partials_hbm, tab_v, xrow_v, vals_v, sem, sem2):
    w = _wid()

    for f in range(NUM_FIELDS):

        @pl.when(w == f)
        def _(f=f):
            src = (p0_hbm, p1_hbm)[f // FPP]
            pltpu.async_copy(
                src.at[pl.ds((f % FPP) * FIELD_SIZE, FIELD_SIZE)], tab_v, sem2
            )
            cps = [
                pltpu.async_copy(
                    xt_hbm.at[t, f], xrow_v.at[pl.ds(t * B_PER_W, B_PER_W)], sem
                )
                for t in range(NW)
            ]
            for cp in cps:
                cp.wait()
            pltpu.make_async_copy(
                src.at[pl.ds((f % FPP) * FIELD_SIZE, FIELD_SIZE)], tab_v, sem2
            ).wait()

            def gath(i, carry):
                sl = pl.ds(i * L, L)
                vals_v[sl] = plsc.load_gather(tab_v, [xrow_v[sl]])
                return carry

            lax.fori_loop(0, BATCH // L, gath, 0)
            pltpu.sync_copy(vals_v, partials_hbm.at[f])


def _reduce_body(partials_hbm, bias_hbm, out_hbm, pcols_v, bias_v, out_v):
    w = _wid()
    base = pl.multiple_of(w * B_PER_W, 8)
    pltpu.sync_copy(partials_hbm.at[:, pl.ds(base, B_PER_W)], pcols_v)
    pltpu.sync_copy(bias_hbm, bias_v)
    iota = lax.iota(jnp.int32, L)
    bias_b = plsc.load_gather(bias_v, [iota * 0])

    def red(q, carry):
        acc = bias_b
        for f in range(NUM_FIELDS):
            acc = acc + pcols_v[f, pl.ds(q * L, L)]
        out_v[pl.ds(q * L, L)] = acc
        return carry

    lax.fori_loop(0, B_PER_W // L, red, 0)
    pltpu.sync_copy(out_v, out_hbm.at[pl.ds(base, B_PER_W)])


@jax.jit
def _run(x, bias, p0, p1):
    k1 = pl.kernel(
        _transpose_body,
        out_type=jax.ShapeDtypeStruct((NW, NUM_FIELDS, B_PER_W), jnp.int32),
        mesh=_mesh(),
        scratch_types=[
            pltpu.VMEM((B_PER_W, NUM_FIELDS), jnp.int32),
            pltpu.VMEM((NUM_FIELDS, B_PER_W), jnp.int32),
        ],
        name="fl_transpose",
        compiler_params=_SC_PARAMS,
    )
    xt = k1(x)

    k2 = pl.kernel(
        _gather_body,
        out_type=jax.ShapeDtypeStruct((NUM_FIELDS, BATCH), jnp.float32),
        mesh=_mesh(),
        scratch_types=[
            pltpu.VMEM((FIELD_SIZE,), jnp.float32),
            pltpu.VMEM((BATCH,), jnp.int32),
            pltpu.VMEM((BATCH,), jnp.float32),
            pltpu.SemaphoreType.DMA,
            pltpu.SemaphoreType.DMA,
        ],
        name="fl_gather",
        compiler_params=_SC_PARAMS,
    )
    partials = k2(p0, p1, xt)

    k3 = pl.kernel(
        _reduce_body,
        out_type=jax.ShapeDtypeStruct((BATCH,), jnp.float32),
        mesh=_mesh(),
        scratch_types=[
            pltpu.VMEM((NUM_FIELDS, B_PER_W), jnp.float32),
            pltpu.VMEM((1,), jnp.float32),
            pltpu.VMEM((B_PER_W,), jnp.float32),
        ],
        name="fl_reduce",
        compiler_params=_SC_PARAMS,
    )
    return k3(partials, bias)


def kernel(x, fc_weight, bias):
    psz = FPP * FIELD_SIZE
    p0 = fc_weight[:psz, 0]
    p1 = fc_weight[psz:, 0]
    out = _run(x.astype(jnp.int32), bias.astype(jnp.float32), p0, p1)
    return out.reshape(BATCH, 1)


# speedup vs baseline: 1.5196x; 1.0992x over previous
import jax
import jax.numpy as jnp
from jax import lax
from jax.experimental import pallas as pl
from jax.experimental.pallas import tpu as pltpu
from jax.experimental.pallas import tpu_sc as plsc

NUM_FIELDS = 26
FIELD_SIZE = 40000
BATCH = 16384
L = 16
NC = 2
NS = 16
NW = NC * NS
B_PER_W = BATCH // NW
NPIECE = 2
FPP = NUM_FIELDS // NPIECE

_SC_PARAMS = pltpu.CompilerParams(needs_layout_passes=False)


def _mesh():
    return plsc.VectorSubcoreMesh(core_axis_name="c", subcore_axis_name="s")


def _wid():
    return lax.axis_index("s") * NC + lax.axis_index("c")


def _transpose_body(x_hbm, xt_hbm, xb_v, xtl_v):
    w = _wid()
    rowbase = pl.multiple_of(w * B_PER_W, 8)
    pltpu.sync_copy(x_hbm.at[pl.ds(rowbase, B_PER_W), :], xb_v)
    iota = lax.iota(jnp.int32, L)

    def chunk(g, carry):
        rows = iota + g * L
        for f in range(NUM_FIELDS):
            xtl_v[f, pl.ds(g * L, L)] = plsc.load_gather(xb_v, [rows, iota * 0 + f])
        return carry

    lax.fori_loop(0, B_PER_W // L, chunk, 0)
    pltpu.sync_copy(xtl_v, xt_hbm.at[:, pl.ds(rowbase, B_PER_W)])


def _gather_body(p0_hbm, p1_hbm, xt_hbm, partials_hbm, tab_v, xrow_v, vals_v, sem, sem2):
    w = _wid()

    for f in range(NUM_FIELDS):

        @pl.when(w == f)
        def _(f=f):
            src = (p0_hbm, p1_hbm)[f // FPP]
            pltpu.async_copy(
                src.at[pl.ds((f % FPP) * FIELD_SIZE, FIELD_SIZE)], tab_v, sem2
            )
            pltpu.async_copy(xt_hbm.at[f], xrow_v, sem)
            pltpu.make_async_copy(xt_hbm.at[f], xrow_v, sem).wait()
            pltpu.make_async_copy(
                src.at[pl.ds((f % FPP) * FIELD_SIZE, FIELD_SIZE)], tab_v, sem2
            ).wait()

            def gath(i, carry):
                sl = pl.ds(i * L, L)
                vals_v[sl] = plsc.load_gather(tab_v, [xrow_v[sl]])
                return carry

            lax.fori_loop(0, BATCH // L, gath, 0)
            pltpu.sync_copy(vals_v, partials_hbm.at[f])


def _reduce_body(partials_hbm, bias_hbm, out_hbm, pcols_v, bias_v, out_v):
    w = _wid()
    base = pl.multiple_of(w * B_PER_W, 8)
    pltpu.sync_copy(partials_hbm.at[:, pl.ds(base, B_PER_W)], pcols_v)
    pltpu.sync_copy(bias_hbm, bias_v)
    iota = lax.iota(jnp.int32, L)
    bias_b = plsc.load_gather(bias_v, [iota * 0])

    def red(q, carry):
        acc = bias_b
        for f in range(NUM_FIELDS):
            acc = acc + pcols_v[f, pl.ds(q * L, L)]
        out_v[pl.ds(q * L, L)] = acc
        return carry

    lax.fori_loop(0, B_PER_W // L, red, 0)
    pltpu.sync_copy(out_v, out_hbm.at[pl.ds(base, B_PER_W)])


@jax.jit
def _run(x, bias, p0, p1):
    k1 = pl.kernel(
        _transpose_body,
        out_type=jax.ShapeDtypeStruct((NUM_FIELDS, BATCH), jnp.int32),
        mesh=_mesh(),
        scratch_types=[
            pltpu.VMEM((B_PER_W, NUM_FIELDS), jnp.int32),
            pltpu.VMEM((NUM_FIELDS, B_PER_W), jnp.int32),
        ],
        name="fl_transpose",
        compiler_params=_SC_PARAMS,
    )
    xt = k1(x)

    k2 = pl.kernel(
        _gather_body,
        out_type=jax.ShapeDtypeStruct((NUM_FIELDS, BATCH), jnp.float32),
        mesh=_mesh(),
        scratch_types=[
            pltpu.VMEM((FIELD_SIZE,), jnp.float32),
            pltpu.VMEM((BATCH,), jnp.int32),
            pltpu.VMEM((BATCH,), jnp.float32),
            pltpu.SemaphoreType.DMA,
            pltpu.SemaphoreType.DMA,
        ],
        name="fl_gather",
        compiler_params=_SC_PARAMS,
    )
    partials = k2(p0, p1, xt)

    k3 = pl.kernel(
        _reduce_body,
        out_type=jax.ShapeDtypeStruct((BATCH,), jnp.float32),
        mesh=_mesh(),
        scratch_types=[
            pltpu.VMEM((NUM_FIELDS, B_PER_W), jnp.float32),
            pltpu.VMEM((1,), jnp.float32),
            pltpu.VMEM((B_PER_W,), jnp.float32),
        ],
        name="fl_reduce",
        compiler_params=_SC_PARAMS,
    )
    return k3(partials, bias)


def kernel(x, fc_weight, bias):
    psz = FPP * FIELD_SIZE
    p0 = fc_weight[:psz, 0]
    p1 = fc_weight[psz:, 0]
    out = _run(x.astype(jnp.int32), bias.astype(jnp.float32), p0, p1)
    return out.reshape(BATCH, 1)
